# baseline (speedup 1.0000x reference)
import functools

import jax
import jax.numpy as jnp
from jax import lax
from jax.experimental import pallas as pl
from jax.experimental.pallas import tpu as pltpu
from jax.experimental.pallas import tpu_sc as plsc

IDX_CHUNK = 128


@functools.lru_cache(maxsize=None)
def _make_gather(B: int, E: int):
    info = plsc.get_sparse_core_info()
    nc, ns = info.num_cores, info.num_subcores
    nw = nc * ns
    b_per_w = B // nw
    assert B % (8 * nw) == 0
    n_chunks = b_per_w // IDX_CHUNK
    assert b_per_w % IDX_CHUNK == 0

    mesh = plsc.VectorSubcoreMesh(core_axis_name="c", subcore_axis_name="s")

    @functools.partial(
        pl.kernel,
        mesh=mesh,
        out_type=jax.ShapeDtypeStruct((4, B, E), jnp.float32),
        scratch_types=[
            pltpu.VMEM((b_per_w,), jnp.int32),
            pltpu.VMEM((b_per_w, E), jnp.float32),
            pltpu.SemaphoreType.DMA,
        ],
        compiler_params=pltpu.CompilerParams(use_tc_tiling_on_sc=False),
    )
    def gather_k(oi, pi, ri, vi, ot, pt, rt, vt, out, idx_v, rows_v, sem):
        wid = lax.axis_index("s") * nc + lax.axis_index("c")
        base = wid * b_per_w
        for t, (ih, th) in enumerate(((oi, ot), (pi, pt), (ri, rt), (vi, vt))):
            pltpu.sync_copy(ih.at[pl.ds(base, b_per_w)], idx_v)
            cps = []
            for j in range(n_chunks):
                cps.append(
                    pltpu.async_copy(
                        th.at[idx_v.at[pl.ds(j * IDX_CHUNK, IDX_CHUNK)]],
                        rows_v.at[pl.ds(j * IDX_CHUNK, IDX_CHUNK)],
                        sem,
                    )
                )
            for cp in cps:
                cp.wait()
            pltpu.sync_copy(rows_v, out.at[t].at[pl.ds(base, b_per_w)])

    return gather_k


@functools.lru_cache(maxsize=None)
def _make_format(N: int, E: int):
    info = plsc.get_sparse_core_info()
    nc, ns = info.num_cores, info.num_subcores
    nw = nc * ns
    NTC = N // (4 * E)
    tail = N - NTC * 4 * E
    nb = (NTC + nw - 1) // nw
    mesh = plsc.VectorSubcoreMesh(core_axis_name="c", subcore_axis_name="s")

    @functools.partial(
        pl.kernel,
        mesh=mesh,
        out_type=jax.ShapeDtypeStruct((N // 4, 4 * E), jnp.float32),
        scratch_types=[
            pltpu.VMEM((E, 128), jnp.float32),
            pltpu.VMEM((E, 128), jnp.float32),
            pltpu.SemaphoreType.DMA,
        ],
        compiler_params=pltpu.CompilerParams(
            use_tc_tiling_on_sc=True, needs_layout_passes=False
        ),
    )
    def fmt(tT, out, fm, sm, sem):
        wid = lax.axis_index("s") * nc + lax.axis_index("c")
        iota = lax.iota(jnp.int32, 16)
        idx_f = [iota, iota + 16]

        def do_stage(col0):
            cps = [
                pltpu.async_copy(
                    tT.at[pl.ds(8 * a, 8), pl.ds(col0, 128)],
                    fm.at[pl.ds(8 * a, 8), :],
                    sem,
                )
                for a in range(E // 8)
            ]
            for cp in cps:
                cp.wait()

            def body(i, carry):
                rv = jnp.full((16,), i, jnp.int32)
                for u in range(4):
                    sv = jnp.full((16,), 4 * i + u, jnp.int32)
                    for h in range(2):
                        v = plsc.load_gather(fm, [idx_f[h], sv])
                        plsc.store_scatter(sm, [rv, iota + (E * u + 16 * h)], v)
                return carry

            lax.fori_loop(0, 32, body, 0)

        for k in range(nb):
            b = wid + nw * k

            @pl.when(b < NTC)
            def _():
                do_stage(pl.multiple_of(128 * b, 128))
                pltpu.sync_copy(sm, out.at[pl.ds(pl.multiple_of(32 * b, 32), E)])

        if tail:

            @pl.when(wid == NTC % nw)
            def _():
                bt = wid - (NTC % nw) + NTC
                do_stage(pl.multiple_of(128 * bt, 128))
                pltpu.sync_copy(
                    sm.at[pl.ds(0, tail // 4)],
                    out.at[pl.ds(32 * NTC, tail // 4)],
                )

    return fmt


def _make_assemble_body(BM, E):
    def body(g_ref, fa_ref, cf_ref, fw_ref, fb_ref, cw_ref, cb_ref, out_ref):
        gs = [g_ref[t] for t in range(4)]
        rows = []
        for r in range(4):
            lo, hi = E * r, E * r + E
            rows.append(jnp.concatenate([gs[t][:, lo:hi] for t in range(4)], axis=1))
        gblk = jnp.stack(rows, axis=1).reshape(BM, 4 * E)
        gT = jnp.transpose(gblk)
        flT = (
            lax.dot_general(
                fw_ref[...],
                fa_ref[...],
                (((0,), (1,)), ((), ())),
                preferred_element_type=jnp.float32,
            )
            + fb_ref[...]
        )
        ctT = (
            lax.dot_general(
                cw_ref[...],
                cf_ref[...],
                (((0,), (0,)), ((), ())),
                preferred_element_type=jnp.float32,
            )
            + cb_ref[...]
        )
        out_ref[...] = jnp.concatenate([gT, flT, ctT], axis=0)

    return body


@functools.lru_cache(maxsize=None)
def _make_assemble(B: int, E: int, F: int, BM: int):
    BMq = BM // 4
    return pl.pallas_call(
        _make_assemble_body(BM, E),
        grid=(B // BM,),
        in_specs=[
            pl.BlockSpec((4, BMq, 4 * E), lambda i: (0, i, 0)),
            pl.BlockSpec((BM, F), lambda i: (i, 0)),
            pl.BlockSpec((3, BM), lambda i: (0, i)),
            pl.BlockSpec((F, E), lambda i: (0, 0)),
            pl.BlockSpec((E, 1), lambda i: (0, 0)),
            pl.BlockSpec((3, E), lambda i: (0, 0)),
            pl.BlockSpec((E, 1), lambda i: (0, 0)),
        ],
        out_specs=pl.BlockSpec((6 * E, BM), lambda i: (0, i)),
        out_shape=jax.ShapeDtypeStruct((6 * E, B), jnp.float32),
    )


def kernel(
    origin,
    process,
    roast_level,
    variety,
    flavors,
    target_finish_temp,
    altitude,
    bean_density,
    origin_table,
    process_table,
    roast_table,
    variety_table,
    flavor_W,
    flavor_b,
    cont_W,
    cont_b,
):
    B, F = flavors.shape
    E = origin_table.shape[1]
    oi = origin.reshape(B).astype(jnp.int32)
    pi = process.reshape(B).astype(jnp.int32)
    ri = roast_level.reshape(B).astype(jnp.int32)
    vi = variety.reshape(B).astype(jnp.int32)

    def _row_major(t):
        n = t.shape[0]
        packed = _make_format(n, E)(t.T)
        return jnp.reshape(packed, t.shape)

    g = _make_gather(B, E)(
        oi,
        pi,
        ri,
        vi,
        _row_major(origin_table),
        process_table,
        roast_table,
        _row_major(variety_table),
    )
    g = jnp.reshape(g, (4, B // 4, 4 * E))
    cfT = jnp.concatenate(
        [target_finish_temp.T, altitude.T, bean_density.T], axis=0
    )
    outT = _make_assemble(B, E, F, 1024)(
        g,
        flavors,
        cfT,
        flavor_W,
        flavor_b.reshape(E, 1),
        cont_W,
        cont_b.reshape(E, 1),
    )
    return outT.T


# speedup vs baseline: 1.8576x; 1.8576x over previous
import functools

import jax
import jax.numpy as jnp
from jax import lax
from jax.experimental import pallas as pl
from jax.experimental.pallas import tpu as pltpu
from jax.experimental.pallas import tpu_sc as plsc

IDX_CHUNK = 128


@functools.lru_cache(maxsize=None)
def _make_gather(B: int, E: int):
    info = plsc.get_sparse_core_info()
    nc, ns = info.num_cores, info.num_subcores
    nw = nc * ns
    b_per_w = B // nw
    assert B % (8 * nw) == 0
    n_chunks = b_per_w // IDX_CHUNK
    assert b_per_w % IDX_CHUNK == 0

    mesh = plsc.VectorSubcoreMesh(core_axis_name="c", subcore_axis_name="s")

    @functools.partial(
        pl.kernel,
        mesh=mesh,
        out_type=jax.ShapeDtypeStruct((4, B, E), jnp.float32),
        scratch_types=[
            pltpu.VMEM((b_per_w,), jnp.int32),
            pltpu.VMEM((b_per_w, E), jnp.float32),
            pltpu.SemaphoreType.DMA,
        ],
        compiler_params=pltpu.CompilerParams(use_tc_tiling_on_sc=False),
    )
    def gather_k(oi, pi, ri, vi, ot, pt, rt, vt, out, idx_v, rows_v, sem):
        wid = lax.axis_index("s") * nc + lax.axis_index("c")
        base = wid * b_per_w
        for t, (ih, th) in enumerate(((oi, ot), (pi, pt), (ri, rt), (vi, vt))):
            pltpu.sync_copy(ih.at[pl.ds(base, b_per_w)], idx_v)
            cps = []
            for j in range(n_chunks):
                cps.append(
                    pltpu.async_copy(
                        th.at[idx_v.at[pl.ds(j * IDX_CHUNK, IDX_CHUNK)]],
                        rows_v.at[pl.ds(j * IDX_CHUNK, IDX_CHUNK)],
                        sem,
                    )
                )
            for cp in cps:
                cp.wait()
            pltpu.sync_copy(rows_v, out.at[t].at[pl.ds(base, b_per_w)])

    return gather_k


def _format_tc_body(x_ref, o_ref):
    x = x_ref[...]
    cw = x.shape[1]
    xT = jnp.transpose(x)
    xr = jnp.reshape(xT, (cw // 4, 4, x.shape[0]))
    o_ref[...] = jnp.concatenate([xr[:, u, :] for u in range(4)], axis=1)


@functools.lru_cache(maxsize=None)
def _make_format_tc(N: int, E: int, CW: int):
    return pl.pallas_call(
        _format_tc_body,
        grid=(pl.cdiv(N, CW),),
        in_specs=[pl.BlockSpec((E, CW), lambda i: (0, i))],
        out_specs=pl.BlockSpec((CW // 4, 4 * E), lambda i: (i, 0)),
        out_shape=jax.ShapeDtypeStruct((N // 4, 4 * E), jnp.float32),
    )


@functools.lru_cache(maxsize=None)
def _make_format(N: int, E: int):
    info = plsc.get_sparse_core_info()
    nc, ns = info.num_cores, info.num_subcores
    nw = nc * ns
    NTC = N // (4 * E)
    tail = N - NTC * 4 * E
    nb = (NTC + nw - 1) // nw
    mesh = plsc.VectorSubcoreMesh(core_axis_name="c", subcore_axis_name="s")

    @functools.partial(
        pl.kernel,
        mesh=mesh,
        out_type=jax.ShapeDtypeStruct((N // 4, 4 * E), jnp.float32),
        scratch_types=[
            pltpu.VMEM((nb + 1, E, 128), jnp.float32),
            pltpu.VMEM((E, 128), jnp.float32),
            pltpu.SemaphoreType.DMA,
        ],
        compiler_params=pltpu.CompilerParams(
            use_tc_tiling_on_sc=True, needs_layout_passes=False
        ),
    )
    def fmt(tT, out, fm, sm, sem):
        wid = lax.axis_index("s") * nc + lax.axis_index("c")
        iota = lax.iota(jnp.int32, 16)
        idx_f = [iota, iota + 16]

        def fire(k, col0):
            for a in range(E // 8):
                pltpu.async_copy(
                    tT.at[pl.ds(8 * a, 8), pl.ds(col0, 128)],
                    fm.at[k].at[pl.ds(8 * a, 8), :],
                    sem,
                )

        def drain(k):
            for a in range(E // 8):
                pltpu.make_async_copy(
                    tT.at[pl.ds(8 * a, 8), pl.ds(0, 128)],
                    fm.at[k].at[pl.ds(8 * a, 8), :],
                    sem,
                ).wait()

        def transpose_to_sm(k):
            def body(i, carry):
                rv = jnp.full((16,), i, jnp.int32)
                for u in range(4):
                    sv = jnp.full((16,), 4 * i + u, jnp.int32)
                    for h in range(2):
                        v = plsc.load_gather(fm.at[k], [idx_f[h], sv])
                        plsc.store_scatter(sm, [rv, iota + (E * u + 16 * h)], v)
                return carry

            lax.fori_loop(0, 32, body, 0)

        for k in range(nb):
            b = wid + nw * k

            @pl.when(b < NTC)
            def _():
                fire(k, pl.multiple_of(128 * b, 128))

        if tail:

            @pl.when(wid == NTC % nw)
            def _():
                bt = wid - (NTC % nw) + NTC
                fire(nb, pl.multiple_of(128 * bt, 128))

        for k in range(nb):
            b = wid + nw * k

            @pl.when(b < NTC)
            def _():
                drain(k)

        if tail:

            @pl.when(wid == NTC % nw)
            def _():
                drain(nb)

        for k in range(nb):
            b = wid + nw * k

            @pl.when(b < NTC)
            def _():
                transpose_to_sm(k)
                pltpu.sync_copy(sm, out.at[pl.ds(pl.multiple_of(32 * b, 32), E)])

        if tail:

            @pl.when(wid == NTC % nw)
            def _():
                transpose_to_sm(nb)
                pltpu.sync_copy(
                    sm.at[pl.ds(0, tail // 4)],
                    out.at[pl.ds(32 * NTC, tail // 4)],
                )

    return fmt


def _make_assemble_body(BM, E):
    def body(g_ref, fa_ref, cf_ref, fw_ref, fb_ref, cw_ref, cb_ref, out_ref):
        gs = [g_ref[t] for t in range(4)]
        rows = []
        for r in range(4):
            lo, hi = E * r, E * r + E
            rows.append(jnp.concatenate([gs[t][:, lo:hi] for t in range(4)], axis=1))
        gblk = jnp.stack(rows, axis=1).reshape(BM, 4 * E)
        gT = jnp.transpose(gblk)
        flT = (
            lax.dot_general(
                fw_ref[...],
                fa_ref[...],
                (((0,), (1,)), ((), ())),
                preferred_element_type=jnp.float32,
            )
            + fb_ref[...]
        )
        ctT = (
            lax.dot_general(
                cw_ref[...],
                cf_ref[...],
                (((0,), (0,)), ((), ())),
                preferred_element_type=jnp.float32,
            )
            + cb_ref[...]
        )
        out_ref[...] = jnp.concatenate([gT, flT, ctT], axis=0)

    return body


@functools.lru_cache(maxsize=None)
def _make_assemble(B: int, E: int, F: int, BM: int):
    BMq = BM // 4
    return pl.pallas_call(
        _make_assemble_body(BM, E),
        grid=(B // BM,),
        in_specs=[
            pl.BlockSpec((4, BMq, 4 * E), lambda i: (0, i, 0)),
            pl.BlockSpec((BM, F), lambda i: (i, 0)),
            pl.BlockSpec((3, BM), lambda i: (0, i)),
            pl.BlockSpec((F, E), lambda i: (0, 0)),
            pl.BlockSpec((E, 1), lambda i: (0, 0)),
            pl.BlockSpec((3, E), lambda i: (0, 0)),
            pl.BlockSpec((E, 1), lambda i: (0, 0)),
        ],
        out_specs=pl.BlockSpec((6 * E, BM), lambda i: (0, i)),
        out_shape=jax.ShapeDtypeStruct((6 * E, B), jnp.float32),
    )


def kernel(
    origin,
    process,
    roast_level,
    variety,
    flavors,
    target_finish_temp,
    altitude,
    bean_density,
    origin_table,
    process_table,
    roast_table,
    variety_table,
    flavor_W,
    flavor_b,
    cont_W,
    cont_b,
):
    B, F = flavors.shape
    E = origin_table.shape[1]
    oi = origin.reshape(B).astype(jnp.int32)
    pi = process.reshape(B).astype(jnp.int32)
    ri = roast_level.reshape(B).astype(jnp.int32)
    vi = variety.reshape(B).astype(jnp.int32)

    def _row_major_sc(t):
        n = t.shape[0]
        packed = _make_format(n, E)(t.T)
        return jnp.reshape(packed, t.shape)

    def _row_major_tc(t):
        n = t.shape[0]
        packed = _make_format_tc(n, E, 2560)(t.T)
        return jnp.reshape(packed, t.shape)

    g = _make_gather(B, E)(
        oi,
        pi,
        ri,
        vi,
        _row_major_tc(origin_table),
        process_table,
        roast_table,
        _row_major_sc(variety_table),
    )
    g = jnp.reshape(g, (4, B // 4, 4 * E))
    cfT = jnp.concatenate(
        [target_finish_temp.T, altitude.T, bean_density.T], axis=0
    )
    outT = _make_assemble(B, E, F, 1024)(
        g,
        flavors,
        cfT,
        flavor_W,
        flavor_b.reshape(E, 1),
        cont_W,
        cont_b.reshape(E, 1),
    )
    return outT.T


# speedup vs baseline: 2.1528x; 1.1589x over previous
import functools

import jax
import jax.numpy as jnp
from jax import lax
from jax.experimental import pallas as pl
from jax.experimental.pallas import tpu as pltpu
from jax.experimental.pallas import tpu_sc as plsc

IDX_CHUNK = 128


@functools.lru_cache(maxsize=None)
def _make_gather(B: int, E: int, NO: int, NV: int):
    info = plsc.get_sparse_core_info()
    nc, ns = info.num_cores, info.num_subcores
    nw = nc * ns
    b_per_w = B // nw
    assert B % (8 * nw) == 0
    n_chunks = b_per_w // IDX_CHUNK
    assert b_per_w % IDX_CHUNK == 0
    NSB = B // IDX_CHUNK

    mesh = plsc.VectorSubcoreMesh(core_axis_name="c", subcore_axis_name="s")

    @functools.partial(
        pl.kernel,
        mesh=mesh,
        out_type=(
            jax.ShapeDtypeStruct((E, NSB, IDX_CHUNK), jnp.float32),
            jax.ShapeDtypeStruct((E, NSB, IDX_CHUNK), jnp.float32),
            jax.ShapeDtypeStruct((2, B, E), jnp.float32),
        ),
        scratch_types=[
            pltpu.VMEM((b_per_w,), jnp.int32),
            pltpu.VMEM((n_chunks, E, IDX_CHUNK), jnp.int32),
            pltpu.VMEM((n_chunks, E, 1, IDX_CHUNK), jnp.float32),
            pltpu.VMEM((b_per_w, E), jnp.float32),
            pltpu.SemaphoreType.DMA,
        ],
        compiler_params=pltpu.CompilerParams(use_tc_tiling_on_sc=False),
    )
    def gather_k(
        oi, pi, ri, vi, tfo, pt, rt, tfv, out_o, out_v, out_s, idx_v, idxb, gv, rows_v, sem
    ):
        wid = lax.axis_index("s") * nc + lax.axis_index("c")
        base = wid * b_per_w

        for t, (ih, th) in enumerate(((pi, pt), (ri, rt))):
            pltpu.sync_copy(ih.at[pl.ds(base, b_per_w)], idx_v)
            cps = []
            for j in range(n_chunks):
                cps.append(
                    pltpu.async_copy(
                        th.at[idx_v.at[pl.ds(j * IDX_CHUNK, IDX_CHUNK)]],
                        rows_v.at[pl.ds(j * IDX_CHUNK, IDX_CHUNK)],
                        sem,
                    )
                )
            for cp in cps:
                cp.wait()
            pltpu.sync_copy(rows_v, out_s.at[t].at[pl.ds(base, b_per_w)])

        for ih, tf, ob, nbig in ((oi, tfo, out_o, NO), (vi, tfv, out_v, NV)):
            pltpu.sync_copy(ih.at[pl.ds(base, b_per_w)], idx_v)
            for c in range(n_chunks):
                for h in range(IDX_CHUNK // 16):
                    xv = idx_v[pl.ds(c * IDX_CHUNK + 16 * h, 16)]
                    for f in range(E):
                        idxb.at[c, f][pl.ds(16 * h, 16)] = xv + f * nbig
            cps = []
            for c in range(n_chunks):
                for f in range(E):
                    cps.append(
                        pltpu.async_copy(
                            tf.at[idxb.at[c, f]], gv.at[c, f, 0], sem
                        )
                    )
            for cp in cps:
                cp.wait()
            for c in range(n_chunks):
                sb = n_chunks * wid + c
                pltpu.sync_copy(gv.at[c], ob.at[:, pl.ds(sb, 1), :])

    return gather_k


def _make_assemble_body(BM, E, NSB_BLK):
    def body(go_ref, gv_ref, gs_ref, fa_ref, cf_ref, fw_ref, fb_ref, cw_ref, cb_ref, out_ref):
        o_band = jnp.concatenate(
            [go_ref[:, sb, :] for sb in range(NSB_BLK)], axis=1
        )
        v_band = jnp.concatenate(
            [gv_ref[:, sb, :] for sb in range(NSB_BLK)], axis=1
        )
        gs = [gs_ref[t] for t in range(2)]
        rows = []
        for r in range(4):
            lo, hi = E * r, E * r + E
            rows.append(
                jnp.concatenate([gs[0][:, lo:hi], gs[1][:, lo:hi]], axis=1)
            )
        sblk = jnp.stack(rows, axis=1).reshape(BM, 2 * E)
        sT = jnp.transpose(sblk)
        flT = (
            lax.dot_general(
                fw_ref[...],
                fa_ref[...],
                (((0,), (1,)), ((), ())),
                preferred_element_type=jnp.float32,
            )
            + fb_ref[...]
        )
        ctT = (
            lax.dot_general(
                cw_ref[...],
                cf_ref[...],
                (((0,), (0,)), ((), ())),
                preferred_element_type=jnp.float32,
            )
            + cb_ref[...]
        )
        out_ref[...] = jnp.concatenate(
            [o_band, sT[0:E], sT[E : 2 * E], v_band, flT, ctT], axis=0
        )

    return body


@functools.lru_cache(maxsize=None)
def _make_assemble(B: int, E: int, F: int, BM: int):
    BMq = BM // 4
    NSB_BLK = BM // 128
    return pl.pallas_call(
        _make_assemble_body(BM, E, NSB_BLK),
        grid=(B // BM,),
        in_specs=[
            pl.BlockSpec((E, NSB_BLK, 128), lambda i: (0, i, 0)),
            pl.BlockSpec((E, NSB_BLK, 128), lambda i: (0, i, 0)),
            pl.BlockSpec((2, BMq, 4 * E), lambda i: (0, i, 0)),
            pl.BlockSpec((BM, F), lambda i: (i, 0)),
            pl.BlockSpec((3, BM), lambda i: (0, i)),
            pl.BlockSpec((F, E), lambda i: (0, 0)),
            pl.BlockSpec((E, 1), lambda i: (0, 0)),
            pl.BlockSpec((3, E), lambda i: (0, 0)),
            pl.BlockSpec((E, 1), lambda i: (0, 0)),
        ],
        out_specs=pl.BlockSpec((6 * E, BM), lambda i: (0, i)),
        out_shape=jax.ShapeDtypeStruct((6 * E, B), jnp.float32),
    )


def kernel(
    origin,
    process,
    roast_level,
    variety,
    flavors,
    target_finish_temp,
    altitude,
    bean_density,
    origin_table,
    process_table,
    roast_table,
    variety_table,
    flavor_W,
    flavor_b,
    cont_W,
    cont_b,
):
    B, F = flavors.shape
    E = origin_table.shape[1]

    oi = origin.reshape(B).astype(jnp.int32)
    pi = process.reshape(B).astype(jnp.int32)
    ri = roast_level.reshape(B).astype(jnp.int32)
    vi = variety.reshape(B).astype(jnp.int32)
    tfo = jnp.reshape(origin_table.T, (-1,))
    tfv = jnp.reshape(variety_table.T, (-1,))
    g_o, g_v, g_s = _make_gather(B, E, origin_table.shape[0], variety_table.shape[0])(
        oi, pi, ri, vi, tfo, process_table, roast_table, tfv
    )
    g_s = jnp.reshape(g_s, (2, B // 4, 4 * E))
    cfT = jnp.concatenate(
        [target_finish_temp.T, altitude.T, bean_density.T], axis=0
    )
    outT = _make_assemble(B, E, F, 1024)(
        g_o,
        g_v,
        g_s,
        flavors,
        cfT,
        flavor_W,
        flavor_b.reshape(E, 1),
        cont_W,
        cont_b.reshape(E, 1),
    )
    return outT.T


# speedup vs baseline: 2.2750x; 1.0567x over previous
import functools

import jax
import jax.numpy as jnp
from jax import lax
from jax.experimental import pallas as pl
from jax.experimental.pallas import tpu as pltpu
from jax.experimental.pallas import tpu_sc as plsc

IDX_CHUNK = 128


@functools.lru_cache(maxsize=None)
def _make_gather(B: int, E: int, NO: int, NV: int):
    info = plsc.get_sparse_core_info()
    nc, ns = info.num_cores, info.num_subcores
    nw = nc * ns
    b_per_w = B // nw
    assert B % (8 * nw) == 0
    n_chunks = b_per_w // IDX_CHUNK
    assert b_per_w % IDX_CHUNK == 0
    NSB = B // IDX_CHUNK

    mesh = plsc.VectorSubcoreMesh(core_axis_name="c", subcore_axis_name="s")

    @functools.partial(
        pl.kernel,
        mesh=mesh,
        out_type=jax.ShapeDtypeStruct((2, B, E), jnp.float32),
        scratch_types=[
            pltpu.VMEM((b_per_w,), jnp.int32),
            pltpu.VMEM((b_per_w, E), jnp.float32),
            pltpu.SemaphoreType.DMA,
        ],
        compiler_params=pltpu.CompilerParams(use_tc_tiling_on_sc=False),
    )
    def gather_small(pi, ri, pt, rt, out_s, idx_v, rows_v, sem):
        wid = lax.axis_index("s") * nc + lax.axis_index("c")
        base = wid * b_per_w

        for t, (ih, th) in enumerate(((pi, pt), (ri, rt))):
            pltpu.sync_copy(ih.at[pl.ds(base, b_per_w)], idx_v)
            cps = []
            for j in range(n_chunks):
                cps.append(
                    pltpu.async_copy(
                        th.at[idx_v.at[pl.ds(j * IDX_CHUNK, IDX_CHUNK)]],
                        rows_v.at[pl.ds(j * IDX_CHUNK, IDX_CHUNK)],
                        sem,
                    )
                )
            for cp in cps:
                cp.wait()
            pltpu.sync_copy(rows_v, out_s.at[t].at[pl.ds(base, b_per_w)])

    def make_big(nbig):
        @functools.partial(
            pl.kernel,
            mesh=mesh,
            out_type=jax.ShapeDtypeStruct((E, NSB, IDX_CHUNK), jnp.float32),
            scratch_types=[
                pltpu.VMEM((b_per_w,), jnp.int32),
                pltpu.VMEM((n_chunks, E, IDX_CHUNK), jnp.int32),
                pltpu.VMEM((n_chunks, E, 1, IDX_CHUNK), jnp.float32),
                pltpu.SemaphoreType.DMA,
            ],
            compiler_params=pltpu.CompilerParams(use_tc_tiling_on_sc=False),
        )
        def gather_big(ih, tf, ob, idx_v, idxb, gv, sem):
            wid = lax.axis_index("s") * nc + lax.axis_index("c")
            base = wid * b_per_w
            pltpu.sync_copy(ih.at[pl.ds(base, b_per_w)], idx_v)
            for c in range(n_chunks):
                for h in range(IDX_CHUNK // 16):
                    xv = idx_v[pl.ds(c * IDX_CHUNK + 16 * h, 16)]
                    for f in range(E):
                        idxb.at[c, f][pl.ds(16 * h, 16)] = xv + f * nbig
            cps = []
            for c in range(n_chunks):
                for f in range(E):
                    cps.append(
                        pltpu.async_copy(tf.at[idxb.at[c, f]], gv.at[c, f, 0], sem)
                    )
            for cp in cps:
                cp.wait()
            for c in range(n_chunks):
                sb = n_chunks * wid + c
                pltpu.sync_copy(gv.at[c], ob.at[:, pl.ds(sb, 1), :])

        return gather_big

    return gather_small, make_big(NO), make_big(NV)


def _make_assemble_body(BM, E, NSB_BLK):
    def body(go_ref, gv_ref, gs_ref, fa_ref, cf_ref, fw_ref, fb_ref, cw_ref, cb_ref, out_ref):
        o_band = jnp.concatenate(
            [go_ref[:, sb, :] for sb in range(NSB_BLK)], axis=1
        )
        v_band = jnp.concatenate(
            [gv_ref[:, sb, :] for sb in range(NSB_BLK)], axis=1
        )
        gs = [gs_ref[t] for t in range(2)]
        rows = []
        for r in range(4):
            lo, hi = E * r, E * r + E
            rows.append(
                jnp.concatenate([gs[0][:, lo:hi], gs[1][:, lo:hi]], axis=1)
            )
        sblk = jnp.stack(rows, axis=1).reshape(BM, 2 * E)
        sT = jnp.transpose(sblk)
        flT = (
            lax.dot_general(
                fw_ref[...],
                fa_ref[...],
                (((0,), (1,)), ((), ())),
                preferred_element_type=jnp.float32,
            )
            + fb_ref[...]
        )
        ctT = (
            lax.dot_general(
                cw_ref[...],
                cf_ref[...],
                (((0,), (0,)), ((), ())),
                preferred_element_type=jnp.float32,
            )
            + cb_ref[...]
        )
        out_ref[...] = jnp.concatenate(
            [o_band, sT[0:E], sT[E : 2 * E], v_band, flT, ctT], axis=0
        )

    return body


@functools.lru_cache(maxsize=None)
def _make_assemble(B: int, E: int, F: int, BM: int):
    BMq = BM // 4
    NSB_BLK = BM // 128
    return pl.pallas_call(
        _make_assemble_body(BM, E, NSB_BLK),
        grid=(B // BM,),
        in_specs=[
            pl.BlockSpec((E, NSB_BLK, 128), lambda i: (0, i, 0)),
            pl.BlockSpec((E, NSB_BLK, 128), lambda i: (0, i, 0)),
            pl.BlockSpec((2, BMq, 4 * E), lambda i: (0, i, 0)),
            pl.BlockSpec((BM, F), lambda i: (i, 0)),
            pl.BlockSpec((3, BM), lambda i: (0, i)),
            pl.BlockSpec((F, E), lambda i: (0, 0)),
            pl.BlockSpec((E, 1), lambda i: (0, 0)),
            pl.BlockSpec((3, E), lambda i: (0, 0)),
            pl.BlockSpec((E, 1), lambda i: (0, 0)),
        ],
        out_specs=pl.BlockSpec((6 * E, BM), lambda i: (0, i)),
        out_shape=jax.ShapeDtypeStruct((6 * E, B), jnp.float32),
    )


def kernel(
    origin,
    process,
    roast_level,
    variety,
    flavors,
    target_finish_temp,
    altitude,
    bean_density,
    origin_table,
    process_table,
    roast_table,
    variety_table,
    flavor_W,
    flavor_b,
    cont_W,
    cont_b,
):
    B, F = flavors.shape
    E = origin_table.shape[1]

    oi = origin.reshape(B).astype(jnp.int32)
    pi = process.reshape(B).astype(jnp.int32)
    ri = roast_level.reshape(B).astype(jnp.int32)
    vi = variety.reshape(B).astype(jnp.int32)
    tfo = jnp.reshape(origin_table.T, (-1,))
    tfv = jnp.reshape(variety_table.T, (-1,))
    gather_small, gather_o, gather_v = _make_gather(
        B, E, origin_table.shape[0], variety_table.shape[0]
    )
    g_s = gather_small(pi, ri, process_table, roast_table)
    g_o = gather_o(oi, tfo)
    g_v = gather_v(vi, tfv)
    g_s = jnp.reshape(g_s, (2, B // 4, 4 * E))
    cfT = jnp.concatenate(
        [target_finish_temp.T, altitude.T, bean_density.T], axis=0
    )
    outT = _make_assemble(B, E, F, 1024)(
        g_o,
        g_v,
        g_s,
        flavors,
        cfT,
        flavor_W,
        flavor_b.reshape(E, 1),
        cont_W,
        cont_b.reshape(E, 1),
    )
    return outT.T


# speedup vs baseline: 2.2841x; 1.0040x over previous
import functools

import jax
import jax.numpy as jnp
from jax import lax
from jax.experimental import pallas as pl
from jax.experimental.pallas import tpu as pltpu
from jax.experimental.pallas import tpu_sc as plsc

IDX_CHUNK = 128


@functools.lru_cache(maxsize=None)
def _make_gather(B: int, E: int, NO: int, NV: int):
    info = plsc.get_sparse_core_info()
    nc, ns = info.num_cores, info.num_subcores
    nw = nc * ns
    b_per_w = B // nw
    assert B % (8 * nw) == 0
    n_chunks = b_per_w // IDX_CHUNK
    assert b_per_w % IDX_CHUNK == 0
    NSB = B // IDX_CHUNK

    mesh = plsc.VectorSubcoreMesh(core_axis_name="c", subcore_axis_name="s")

    def big_body(ih, tf, ob, idx_v, idxb, gv, sem, wid, base, nbig):
        pltpu.sync_copy(ih.at[pl.ds(base, b_per_w)], idx_v)
        for c in range(n_chunks):
            for h in range(IDX_CHUNK // 16):
                xv = idx_v[pl.ds(c * IDX_CHUNK + 16 * h, 16)]
                for f in range(E):
                    idxb.at[c, f][pl.ds(16 * h, 16)] = xv + f * nbig
        cps = []
        for c in range(n_chunks):
            for f in range(E):
                cps.append(
                    pltpu.async_copy(tf.at[idxb.at[c, f]], gv.at[c, f, 0], sem)
                )
        for cp in cps:
            cp.wait()
        for c in range(n_chunks):
            sb = n_chunks * wid + c
            pltpu.sync_copy(gv.at[c], ob.at[:, pl.ds(sb, 1), :])

    @functools.partial(
        pl.kernel,
        mesh=mesh,
        out_type=(
            jax.ShapeDtypeStruct((E, NSB, IDX_CHUNK), jnp.float32),
            jax.ShapeDtypeStruct((2, B, E), jnp.float32),
        ),
        scratch_types=[
            pltpu.VMEM((b_per_w,), jnp.int32),
            pltpu.VMEM((n_chunks, E, IDX_CHUNK), jnp.int32),
            pltpu.VMEM((n_chunks, E, 1, IDX_CHUNK), jnp.float32),
            pltpu.VMEM((b_per_w, E), jnp.float32),
            pltpu.SemaphoreType.DMA,
        ],
        compiler_params=pltpu.CompilerParams(use_tc_tiling_on_sc=False),
    )
    def gather_o(oi, pi, ri, tfo, pt, rt, ob, out_s, idx_v, idxb, gv, rows_v, sem):
        wid = lax.axis_index("s") * nc + lax.axis_index("c")
        base = wid * b_per_w

        for t, (ih, th) in enumerate(((pi, pt), (ri, rt))):
            pltpu.sync_copy(ih.at[pl.ds(base, b_per_w)], idx_v)
            cps = []
            for j in range(n_chunks):
                cps.append(
                    pltpu.async_copy(
                        th.at[idx_v.at[pl.ds(j * IDX_CHUNK, IDX_CHUNK)]],
                        rows_v.at[pl.ds(j * IDX_CHUNK, IDX_CHUNK)],
                        sem,
                    )
                )
            for cp in cps:
                cp.wait()
            pltpu.sync_copy(rows_v, out_s.at[t].at[pl.ds(base, b_per_w)])

        big_body(oi, tfo, ob, idx_v, idxb, gv, sem, wid, base, NO)

    @functools.partial(
        pl.kernel,
        mesh=mesh,
        out_type=jax.ShapeDtypeStruct((E, NSB, IDX_CHUNK), jnp.float32),
        scratch_types=[
            pltpu.VMEM((b_per_w,), jnp.int32),
            pltpu.VMEM((n_chunks, E, IDX_CHUNK), jnp.int32),
            pltpu.VMEM((n_chunks, E, 1, IDX_CHUNK), jnp.float32),
            pltpu.SemaphoreType.DMA,
        ],
        compiler_params=pltpu.CompilerParams(use_tc_tiling_on_sc=False),
    )
    def gather_v(vi, tfv, ob, idx_v, idxb, gv, sem):
        wid = lax.axis_index("s") * nc + lax.axis_index("c")
        base = wid * b_per_w
        big_body(vi, tfv, ob, idx_v, idxb, gv, sem, wid, base, NV)

    return gather_o, gather_v


def _make_assemble_body(BM, E, NSB_BLK):
    def body(go_ref, gv_ref, gs_ref, fa_ref, cf_ref, fw_ref, fb_ref, cw_ref, cb_ref, out_ref):
        o_band = jnp.concatenate(
            [go_ref[:, sb, :] for sb in range(NSB_BLK)], axis=1
        )
        v_band = jnp.concatenate(
            [gv_ref[:, sb, :] for sb in range(NSB_BLK)], axis=1
        )
        gs = [gs_ref[t] for t in range(2)]
        rows = []
        for r in range(4):
            lo, hi = E * r, E * r + E
            rows.append(
                jnp.concatenate([gs[0][:, lo:hi], gs[1][:, lo:hi]], axis=1)
            )
        sblk = jnp.stack(rows, axis=1).reshape(BM, 2 * E)
        sT = jnp.transpose(sblk)
        flT = (
            lax.dot_general(
                fw_ref[...],
                fa_ref[...],
                (((0,), (1,)), ((), ())),
                preferred_element_type=jnp.float32,
            )
            + fb_ref[...]
        )
        ctT = (
            lax.dot_general(
                cw_ref[...],
                cf_ref[...],
                (((0,), (0,)), ((), ())),
                preferred_element_type=jnp.float32,
            )
            + cb_ref[...]
        )
        out_ref[...] = jnp.concatenate(
            [o_band, sT[0:E], sT[E : 2 * E], v_band, flT, ctT], axis=0
        )

    return body


@functools.lru_cache(maxsize=None)
def _make_assemble(B: int, E: int, F: int, BM: int):
    BMq = BM // 4
    NSB_BLK = BM // 128
    return pl.pallas_call(
        _make_assemble_body(BM, E, NSB_BLK),
        grid=(B // BM,),
        in_specs=[
            pl.BlockSpec((E, NSB_BLK, 128), lambda i: (0, i, 0)),
            pl.BlockSpec((E, NSB_BLK, 128), lambda i: (0, i, 0)),
            pl.BlockSpec((2, BMq, 4 * E), lambda i: (0, i, 0)),
            pl.BlockSpec((BM, F), lambda i: (i, 0)),
            pl.BlockSpec((3, BM), lambda i: (0, i)),
            pl.BlockSpec((F, E), lambda i: (0, 0)),
            pl.BlockSpec((E, 1), lambda i: (0, 0)),
            pl.BlockSpec((3, E), lambda i: (0, 0)),
            pl.BlockSpec((E, 1), lambda i: (0, 0)),
        ],
        out_specs=pl.BlockSpec((6 * E, BM), lambda i: (0, i)),
        out_shape=jax.ShapeDtypeStruct((6 * E, B), jnp.float32),
    )


def kernel(
    origin,
    process,
    roast_level,
    variety,
    flavors,
    target_finish_temp,
    altitude,
    bean_density,
    origin_table,
    process_table,
    roast_table,
    variety_table,
    flavor_W,
    flavor_b,
    cont_W,
    cont_b,
):
    B, F = flavors.shape
    E = origin_table.shape[1]

    oi = origin.reshape(B).astype(jnp.int32)
    pi = process.reshape(B).astype(jnp.int32)
    ri = roast_level.reshape(B).astype(jnp.int32)
    vi = variety.reshape(B).astype(jnp.int32)
    tfo = jnp.reshape(origin_table.T, (-1,))
    tfv = jnp.reshape(variety_table.T, (-1,))
    gather_o, gather_v = _make_gather(
        B, E, origin_table.shape[0], variety_table.shape[0]
    )
    g_o, g_s = gather_o(oi, pi, ri, tfo, process_table, roast_table)
    g_v = gather_v(vi, tfv)
    g_s = jnp.reshape(g_s, (2, B // 4, 4 * E))
    cfT = jnp.concatenate(
        [target_finish_temp.T, altitude.T, bean_density.T], axis=0
    )
    outT = _make_assemble(B, E, F, 1024)(
        g_o,
        g_v,
        g_s,
        flavors,
        cfT,
        flavor_W,
        flavor_b.reshape(E, 1),
        cont_W,
        cont_b.reshape(E, 1),
    )
    return outT.T


# speedup vs baseline: 2.3700x; 1.0376x over previous
import functools

import jax
import jax.numpy as jnp
from jax import lax
from jax.experimental import pallas as pl
from jax.experimental.pallas import tpu as pltpu
from jax.experimental.pallas import tpu_sc as plsc

IDX_CHUNK = 128


@functools.lru_cache(maxsize=None)
def _make_gather(B: int, E: int, NO: int, NV: int):
    info = plsc.get_sparse_core_info()
    nc, ns = info.num_cores, info.num_subcores
    nw = nc * ns
    b_per_w = B // nw
    assert B % (8 * nw) == 0
    n_chunks = b_per_w // IDX_CHUNK
    assert b_per_w % IDX_CHUNK == 0
    NSB = B // IDX_CHUNK

    mesh = plsc.VectorSubcoreMesh(core_axis_name="c", subcore_axis_name="s")

    def big_body(ih, tf, ob, idx_v, idxb, gv, sem, wid, base, nbig):
        pltpu.sync_copy(ih.at[pl.ds(base, b_per_w)], idx_v)
        for c in range(n_chunks):
            for h in range(IDX_CHUNK // 16):
                xv = idx_v[pl.ds(c * IDX_CHUNK + 16 * h, 16)]
                for f in range(E):
                    idxb.at[c, f][pl.ds(16 * h, 16)] = xv + f * nbig
        cps = []
        for c in range(n_chunks):
            for f in range(E):
                cps.append(
                    pltpu.async_copy(tf.at[idxb.at[c, f]], gv.at[c, f, 0], sem)
                )
        for cp in cps:
            cp.wait()
        for c in range(n_chunks):
            sb = n_chunks * wid + c
            pltpu.sync_copy(gv.at[c], ob.at[:, pl.ds(sb, 1), :])

    @functools.partial(
        pl.kernel,
        mesh=mesh,
        out_type=(
            jax.ShapeDtypeStruct((E, NSB, IDX_CHUNK), jnp.float32),
            jax.ShapeDtypeStruct((2, B, E), jnp.float32),
        ),
        scratch_types=[
            pltpu.VMEM((b_per_w,), jnp.int32),
            pltpu.VMEM((n_chunks, E, IDX_CHUNK), jnp.int32),
            pltpu.VMEM((n_chunks, E, 1, IDX_CHUNK), jnp.float32),
            pltpu.VMEM((b_per_w, E), jnp.float32),
            pltpu.SemaphoreType.DMA,
        ],
        compiler_params=pltpu.CompilerParams(use_tc_tiling_on_sc=False),
    )
    def gather_o(oi, pi, ri, tfo, pt, rt, ob, out_s, idx_v, idxb, gv, rows_v, sem):
        wid = lax.axis_index("s") * nc + lax.axis_index("c")
        base = wid * b_per_w

        for t, (ih, th) in enumerate(((pi, pt), (ri, rt))):
            pltpu.sync_copy(ih.at[pl.ds(base, b_per_w)], idx_v)
            cps = []
            for j in range(n_chunks):
                cps.append(
                    pltpu.async_copy(
                        th.at[idx_v.at[pl.ds(j * IDX_CHUNK, IDX_CHUNK)]],
                        rows_v.at[pl.ds(j * IDX_CHUNK, IDX_CHUNK)],
                        sem,
                    )
                )
            for cp in cps:
                cp.wait()
            pltpu.sync_copy(rows_v, out_s.at[t].at[pl.ds(base, b_per_w)])

        big_body(oi, tfo, ob, idx_v, idxb, gv, sem, wid, base, NO)

    @functools.partial(
        pl.kernel,
        mesh=mesh,
        out_type=jax.ShapeDtypeStruct((E, NSB, IDX_CHUNK), jnp.float32),
        scratch_types=[
            pltpu.VMEM((b_per_w,), jnp.int32),
            pltpu.VMEM((n_chunks, E, IDX_CHUNK), jnp.int32),
            pltpu.VMEM((n_chunks, E, 1, IDX_CHUNK), jnp.float32),
            pltpu.SemaphoreType.DMA,
        ],
        compiler_params=pltpu.CompilerParams(use_tc_tiling_on_sc=False),
    )
    def gather_v(vi, tfv, ob, idx_v, idxb, gv, sem):
        wid = lax.axis_index("s") * nc + lax.axis_index("c")
        base = wid * b_per_w
        big_body(vi, tfv, ob, idx_v, idxb, gv, sem, wid, base, NV)

    return gather_o, gather_v


def _make_assemble_body(BM, E, NSB_BLK):
    def body(go_ref, gv_ref, gs_ref, fa_ref, cf_ref, fw_ref, fb_ref, cw_ref, cb_ref, out_ref):
        o_band = jnp.concatenate(
            [go_ref[:, sb, :] for sb in range(NSB_BLK)], axis=1
        )
        v_band = jnp.concatenate(
            [gv_ref[:, sb, :] for sb in range(NSB_BLK)], axis=1
        )
        gs = [gs_ref[t] for t in range(2)]
        rows = []
        for r in range(4):
            lo, hi = E * r, E * r + E
            rows.append(
                jnp.concatenate([gs[0][:, lo:hi], gs[1][:, lo:hi]], axis=1)
            )
        sblk = jnp.stack(rows, axis=1).reshape(BM, 2 * E)
        sT = jnp.transpose(sblk)
        flT = (
            lax.dot_general(
                fw_ref[...],
                fa_ref[...],
                (((0,), (1,)), ((), ())),
                preferred_element_type=jnp.float32,
            )
            + fb_ref[...]
        )
        ctT = (
            lax.dot_general(
                cw_ref[...],
                cf_ref[...],
                (((0,), (0,)), ((), ())),
                preferred_element_type=jnp.float32,
            )
            + cb_ref[...]
        )
        out_ref[...] = jnp.concatenate(
            [o_band, sT[0:E], sT[E : 2 * E], v_band, flT, ctT], axis=0
        )

    return body


@functools.lru_cache(maxsize=None)
def _make_assemble(B: int, E: int, F: int, BM: int):
    BMq = BM // 4
    NSB_BLK = BM // 128
    return pl.pallas_call(
        _make_assemble_body(BM, E, NSB_BLK),
        grid=(B // BM,),
        in_specs=[
            pl.BlockSpec((E, NSB_BLK, 128), lambda i: (0, i, 0)),
            pl.BlockSpec((E, NSB_BLK, 128), lambda i: (0, i, 0)),
            pl.BlockSpec((2, BMq, 4 * E), lambda i: (0, i, 0)),
            pl.BlockSpec((BM, F), lambda i: (i, 0)),
            pl.BlockSpec((3, BM), lambda i: (0, i)),
            pl.BlockSpec((F, E), lambda i: (0, 0)),
            pl.BlockSpec((E, 1), lambda i: (0, 0)),
            pl.BlockSpec((3, E), lambda i: (0, 0)),
            pl.BlockSpec((E, 1), lambda i: (0, 0)),
        ],
        out_specs=pl.BlockSpec((6 * E, BM), lambda i: (0, i)),
        out_shape=jax.ShapeDtypeStruct((6 * E, B), jnp.float32),
    )


def kernel(
    origin,
    process,
    roast_level,
    variety,
    flavors,
    target_finish_temp,
    altitude,
    bean_density,
    origin_table,
    process_table,
    roast_table,
    variety_table,
    flavor_W,
    flavor_b,
    cont_W,
    cont_b,
):
    B, F = flavors.shape
    E = origin_table.shape[1]

    oi = origin.reshape(B).astype(jnp.int32)
    pi = process.reshape(B).astype(jnp.int32)
    ri = roast_level.reshape(B).astype(jnp.int32)
    vi = variety.reshape(B).astype(jnp.int32)
    tfo = jnp.reshape(origin_table.T, (-1,))
    tfv = jnp.reshape(variety_table.T, (-1,))
    gather_o, gather_v = _make_gather(
        B, E, origin_table.shape[0], variety_table.shape[0]
    )
    g_o, g_s = gather_o(oi, pi, ri, tfo, process_table, roast_table)
    g_v = gather_v(vi, tfv)
    g_s = jnp.reshape(g_s, (2, B // 4, 4 * E))
    cfT = jnp.concatenate(
        [target_finish_temp.T, altitude.T, bean_density.T], axis=0
    )
    outT = _make_assemble(B, E, F, 2048)(
        g_o,
        g_v,
        g_s,
        flavors,
        cfT,
        flavor_W,
        flavor_b.reshape(E, 1),
        cont_W,
        cont_b.reshape(E, 1),
    )
    return outT.T


# speedup vs baseline: 2.3752x; 1.0022x over previous
import functools

import jax
import jax.numpy as jnp
from jax import lax
from jax.experimental import pallas as pl
from jax.experimental.pallas import tpu as pltpu
from jax.experimental.pallas import tpu_sc as plsc

IDX_CHUNK = 128


@functools.lru_cache(maxsize=None)
def _make_gather(B: int, E: int, NO: int, NV: int):
    info = plsc.get_sparse_core_info()
    nc, ns = info.num_cores, info.num_subcores
    nw = nc * ns
    b_per_w = B // nw
    assert B % (8 * nw) == 0
    n_chunks = b_per_w // IDX_CHUNK
    assert b_per_w % IDX_CHUNK == 0
    NSB = B // IDX_CHUNK

    mesh = plsc.VectorSubcoreMesh(core_axis_name="c", subcore_axis_name="s")

    def big_body(ih, tf, ob, idx_v, idxb, gv, sem, wid, base, nbig):
        pltpu.sync_copy(ih.at[pl.ds(base, b_per_w)], idx_v)
        for c in range(n_chunks):
            for h in range(IDX_CHUNK // 16):
                xv = idx_v[pl.ds(c * IDX_CHUNK + 16 * h, 16)]
                for f in range(E):
                    idxb.at[c, f][pl.ds(16 * h, 16)] = xv + f * nbig
        cps = []
        for c in range(n_chunks):
            for f in range(E):
                cps.append(
                    pltpu.async_copy(tf.at[idxb.at[c, f]], gv.at[c, f, 0], sem)
                )
        for cp in cps:
            cp.wait()
        for c in range(n_chunks):
            sb = n_chunks * wid + c
            pltpu.sync_copy(gv.at[c], ob.at[:, pl.ds(sb, 1), :])

    @functools.partial(
        pl.kernel,
        mesh=mesh,
        out_type=(
            jax.ShapeDtypeStruct((E, NSB, IDX_CHUNK), jnp.float32),
            jax.ShapeDtypeStruct((2, B, E), jnp.float32),
        ),
        scratch_types=[
            pltpu.VMEM((b_per_w,), jnp.int32),
            pltpu.VMEM((n_chunks, E, IDX_CHUNK), jnp.int32),
            pltpu.VMEM((n_chunks, E, 1, IDX_CHUNK), jnp.float32),
            pltpu.VMEM((b_per_w, E), jnp.float32),
            pltpu.SemaphoreType.DMA,
        ],
        compiler_params=pltpu.CompilerParams(use_tc_tiling_on_sc=False),
    )
    def gather_o(oi, pi, ri, tfo, pt, rt, ob, out_s, idx_v, idxb, gv, rows_v, sem):
        wid = lax.axis_index("s") * nc + lax.axis_index("c")
        base = wid * b_per_w

        for t, (ih, th) in enumerate(((pi, pt), (ri, rt))):
            pltpu.sync_copy(ih.at[pl.ds(base, b_per_w)], idx_v)
            cps = []
            for j in range(n_chunks):
                cps.append(
                    pltpu.async_copy(
                        th.at[idx_v.at[pl.ds(j * IDX_CHUNK, IDX_CHUNK)]],
                        rows_v.at[pl.ds(j * IDX_CHUNK, IDX_CHUNK)],
                        sem,
                    )
                )
            for cp in cps:
                cp.wait()
            pltpu.sync_copy(rows_v, out_s.at[t].at[pl.ds(base, b_per_w)])

        big_body(oi, tfo, ob, idx_v, idxb, gv, sem, wid, base, NO)

    @functools.partial(
        pl.kernel,
        mesh=mesh,
        out_type=jax.ShapeDtypeStruct((E, NSB, IDX_CHUNK), jnp.float32),
        scratch_types=[
            pltpu.VMEM((b_per_w,), jnp.int32),
            pltpu.VMEM((n_chunks, E, IDX_CHUNK), jnp.int32),
            pltpu.VMEM((n_chunks, E, 1, IDX_CHUNK), jnp.float32),
            pltpu.SemaphoreType.DMA,
        ],
        compiler_params=pltpu.CompilerParams(use_tc_tiling_on_sc=False),
    )
    def gather_v(vi, tfv, ob, idx_v, idxb, gv, sem):
        wid = lax.axis_index("s") * nc + lax.axis_index("c")
        base = wid * b_per_w
        big_body(vi, tfv, ob, idx_v, idxb, gv, sem, wid, base, NV)

    return gather_o, gather_v


def _make_assemble_body(BM, E, NSB_BLK):
    def body(go_ref, gv_ref, gs_ref, fa_ref, cf_ref, fw_ref, fb_ref, cw_ref, cb_ref, out_ref):
        o_band = jnp.concatenate(
            [go_ref[:, sb, :] for sb in range(NSB_BLK)], axis=1
        )
        v_band = jnp.concatenate(
            [gv_ref[:, sb, :] for sb in range(NSB_BLK)], axis=1
        )
        gs = [gs_ref[t] for t in range(2)]
        rows = []
        for r in range(4):
            lo, hi = E * r, E * r + E
            rows.append(
                jnp.concatenate([gs[0][:, lo:hi], gs[1][:, lo:hi]], axis=1)
            )
        sblk = jnp.stack(rows, axis=1).reshape(BM, 2 * E)
        sT = jnp.transpose(sblk)
        flT = (
            lax.dot_general(
                fw_ref[...],
                fa_ref[...],
                (((0,), (1,)), ((), ())),
                preferred_element_type=jnp.float32,
            )
            + fb_ref[...]
        )
        ctT = (
            lax.dot_general(
                cw_ref[...],
                cf_ref[...],
                (((0,), (0,)), ((), ())),
                preferred_element_type=jnp.float32,
            )
            + cb_ref[...]
        )
        out_ref[...] = jnp.concatenate(
            [o_band, sT[0:E], sT[E : 2 * E], v_band, flT, ctT], axis=0
        )

    return body


@functools.lru_cache(maxsize=None)
def _make_assemble(B: int, E: int, F: int, BM: int):
    BMq = BM // 4
    NSB_BLK = BM // 128
    return pl.pallas_call(
        _make_assemble_body(BM, E, NSB_BLK),
        grid=(B // BM,),
        in_specs=[
            pl.BlockSpec((E, NSB_BLK, 128), lambda i: (0, i, 0)),
            pl.BlockSpec((E, NSB_BLK, 128), lambda i: (0, i, 0)),
            pl.BlockSpec((2, BMq, 4 * E), lambda i: (0, i, 0)),
            pl.BlockSpec((BM, F), lambda i: (i, 0)),
            pl.BlockSpec((3, BM), lambda i: (0, i)),
            pl.BlockSpec((F, E), lambda i: (0, 0)),
            pl.BlockSpec((E, 1), lambda i: (0, 0)),
            pl.BlockSpec((3, E), lambda i: (0, 0)),
            pl.BlockSpec((E, 1), lambda i: (0, 0)),
        ],
        out_specs=pl.BlockSpec((6 * E, BM), lambda i: (0, i)),
        out_shape=jax.ShapeDtypeStruct((6 * E, B), jnp.float32),
    )


def kernel(
    origin,
    process,
    roast_level,
    variety,
    flavors,
    target_finish_temp,
    altitude,
    bean_density,
    origin_table,
    process_table,
    roast_table,
    variety_table,
    flavor_W,
    flavor_b,
    cont_W,
    cont_b,
):
    B, F = flavors.shape
    E = origin_table.shape[1]

    oi = origin.reshape(B).astype(jnp.int32)
    pi = process.reshape(B).astype(jnp.int32)
    ri = roast_level.reshape(B).astype(jnp.int32)
    vi = variety.reshape(B).astype(jnp.int32)
    tfo = jnp.reshape(origin_table.T, (-1,))
    tfv = jnp.reshape(variety_table.T, (-1,))
    gather_o, gather_v = _make_gather(
        B, E, origin_table.shape[0], variety_table.shape[0]
    )
    g_o, g_s = gather_o(oi, pi, ri, tfo, process_table, roast_table)
    g_v = gather_v(vi, tfv)
    g_s = jnp.reshape(g_s, (2, B // 4, 4 * E))
    cfT = jnp.concatenate(
        [target_finish_temp.T, altitude.T, bean_density.T], axis=0
    )
    outT = _make_assemble(B, E, F, 4096)(
        g_o,
        g_v,
        g_s,
        flavors,
        cfT,
        flavor_W,
        flavor_b.reshape(E, 1),
        cont_W,
        cont_b.reshape(E, 1),
    )
    return outT.T
